# Initial kernel scaffold; baseline (speedup 1.0000x reference)
import jax
import jax.numpy as jnp
from jax.experimental import pallas as pl


def kernel(x_sotu, x_taxon, params, node_id_sotu, node_id_taxon, edge_index_fwd, edge_index_rev, edge_label_index):
    raise NotImplementedError("write your pallas kernel here")


# speedup vs baseline: 1.0125x; 1.0125x over previous
import functools

import jax
import jax.numpy as jnp
from jax.experimental import pallas as pl


def _mm_kernel(x_ref, w_ref, b_ref, o_ref, *, act):
    y = jnp.dot(x_ref[...], w_ref[...], preferred_element_type=jnp.float32)
    y = y + b_ref[...]
    if act == "relu":
        y = jnp.maximum(y, 0.0)
    o_ref[...] = y


def _mm(x, w, b, act=None, block=2000):
    n, fin = x.shape
    fout = w.shape[1]
    assert n % block == 0, (n, block)
    return pl.pallas_call(
        functools.partial(_mm_kernel, act=act),
        grid=(n // block,),
        in_specs=[
            pl.BlockSpec((block, fin), lambda i: (i, 0)),
            pl.BlockSpec((fin, fout), lambda i: (0, 0)),
            pl.BlockSpec((1, fout), lambda i: (0, 0)),
        ],
        out_specs=pl.BlockSpec((block, fout), lambda i: (i, 0)),
        out_shape=jax.ShapeDtypeStruct((n, fout), jnp.float32),
    )(x, w, b.reshape(1, fout))


def _gat(x_src, x_dst, edge_index, p, num_dst):
    h_src = _mm(x_src, p["W_src"], jnp.zeros_like(p["bias"]))
    h_dst = _mm(x_dst, p["W_dst"], jnp.zeros_like(p["bias"]))
    a_src = (h_src * p["att_src"]).sum(-1)
    a_dst = (h_dst * p["att_dst"]).sum(-1)
    src = edge_index[0]
    dst = edge_index[1]
    e = jax.nn.leaky_relu(a_src[src] + a_dst[dst], 0.2)
    m = jax.ops.segment_max(e, dst, num_segments=num_dst)
    m = jnp.where(jnp.isfinite(m), m, 0.0)
    ex = jnp.exp(e - m[dst])
    denom = jax.ops.segment_sum(ex, dst, num_segments=num_dst)
    alpha = ex / (denom[dst] + 1e-16)
    msg = h_src[src] * alpha[:, None]
    out = jax.ops.segment_sum(msg, dst, num_segments=num_dst)
    return out + p["bias"]


def kernel(x_sotu, x_taxon, params, node_id_sotu, node_id_taxon,
           edge_index_fwd, edge_index_rev, edge_label_index):
    p = params
    hs = x_sotu * p["sotu_lin_W"][0] + p["sotu_lin_b"] + p["sotu_emb"][node_id_sotu]
    ht = x_taxon * p["taxon_lin_W"][0] + p["taxon_lin_b"] + p["taxon_emb"][node_id_taxon]
    zt = jax.nn.relu(_gat(hs, ht, edge_index_fwd, p["c1_fwd"], ht.shape[0]))
    zs = jax.nn.relu(_gat(ht, hs, edge_index_rev, p["c1_rev"], hs.shape[0]))
    zt2 = _gat(zs, zt, edge_index_fwd, p["c2_fwd"], zt.shape[0])
    zs2 = _gat(zt, zs, edge_index_rev, p["c2_rev"], zs.shape[0])
    row = edge_label_index[0]
    col = edge_label_index[1]
    z = jnp.concatenate([zs2[row], zt2[col]], axis=-1)
    z = _mm(z, p["dec_W1"], p["dec_b1"], act="relu")
    z = _mm(z, p["dec_W2"], p["dec_b2"])
    return z.reshape(-1)


# speedup vs baseline: 20.8049x; 20.5485x over previous
import functools

import jax
import jax.numpy as jnp
from jax import lax
from jax.experimental import pallas as pl
from jax.experimental.pallas import tpu as pltpu
from jax.experimental.pallas import tpu_sc as plsc

N = 10000
E = 320000
EL = 100000
H = 128
HA = 144
NW = 32
EPW = E // NW
C = 80
NCH = EPW // C
IB = 5
NP = 10240
ROWS_PW = NP // 16
DC = 80
DPW = 102400 // NW
DNCH = DPW // DC


def _prep_body(x_ref, emb_ref, lw_ref, lb_ref, wa_ref, ea_ref, v_ref,
               haug_ref, a_ref):
    h = x_ref[...] * lw_ref[...] + lb_ref[...] + emb_ref[...]
    haug_ref[...] = jnp.dot(h, wa_ref[...],
                            preferred_element_type=jnp.float32) + ea_ref[...]
    a_ref[...] = jnp.dot(h, v_ref[...], preferred_element_type=jnp.float32)


def _prep(x, emb, lin_w, lin_b, w_aug, e_aug, v, block=2000):
    return pl.pallas_call(
        _prep_body,
        grid=(N // block,),
        in_specs=[
            pl.BlockSpec((block, 1), lambda i: (i, 0)),
            pl.BlockSpec((block, H), lambda i: (i, 0)),
            pl.BlockSpec((1, H), lambda i: (0, 0)),
            pl.BlockSpec((1, H), lambda i: (0, 0)),
            pl.BlockSpec((H, HA), lambda i: (0, 0)),
            pl.BlockSpec((1, HA), lambda i: (0, 0)),
            pl.BlockSpec((H, 2), lambda i: (0, 0)),
        ],
        out_specs=[
            pl.BlockSpec((block, HA), lambda i: (i, 0)),
            pl.BlockSpec((block, 2), lambda i: (i, 0)),
        ],
        out_shape=[
            jax.ShapeDtypeStruct((N, HA), jnp.float32),
            jax.ShapeDtypeStruct((N, 2), jnp.float32),
        ],
    )(x, emb, lin_w, lin_b, w_aug, e_aug, v)


def _mid_body(part_ref, bias_ref, wa_ref, ea_ref, v_ref, haug_ref, a_ref,
              *, relu):
    acc = part_ref[0] + part_ref[1]
    z = acc[:, :H] / (acc[:, H:H + 1] + 1e-16) + bias_ref[...]
    if relu:
        z = jnp.maximum(z, 0.0)
    haug_ref[...] = jnp.dot(z, wa_ref[...],
                            preferred_element_type=jnp.float32) + ea_ref[...]
    a_ref[...] = jnp.dot(z, v_ref[...], preferred_element_type=jnp.float32)


def _mid(part, bias, w_aug, e_aug, v, relu, block=2000):
    wout = w_aug.shape[1]
    return pl.pallas_call(
        functools.partial(_mid_body, relu=relu),
        grid=(N // block,),
        in_specs=[
            pl.BlockSpec((2, block, HA), lambda i: (0, i, 0)),
            pl.BlockSpec((1, H), lambda i: (0, 0)),
            pl.BlockSpec((H, wout), lambda i: (0, 0)),
            pl.BlockSpec((1, wout), lambda i: (0, 0)),
            pl.BlockSpec((H, 2), lambda i: (0, 0)),
        ],
        out_specs=[
            pl.BlockSpec((block, wout), lambda i: (i, 0)),
            pl.BlockSpec((block, 2), lambda i: (i, 0)),
        ],
        out_shape=[
            jax.ShapeDtypeStruct((N, wout), jnp.float32),
            jax.ShapeDtypeStruct((N, 2), jnp.float32),
        ],
    )(part, bias, w_aug, e_aug, v)


def _sc_gat_body(haug_hbm, asrc_hbm, adst_hbm, srcs_hbm, dsts_hbm, out_hbm,
                 srcs_v, dsts_v, asrc_v, adst_v, rows_v, wc_v, acc_sh, sem):
    cid = lax.axis_index("c")
    sid = lax.axis_index("s")
    wid = cid * 16 + sid

    pltpu.sync_copy(asrc_hbm, asrc_v)
    pltpu.sync_copy(adst_hbm, adst_v)

    def zrow(j, carry):
        for k in range(HA // 16):
            rows_v[j, pl.ds(k * 16, 16)] = jnp.zeros((16,), jnp.float32)
        return carry

    lax.fori_loop(0, C, zrow, 0)
    base = sid * ROWS_PW
    for i in range(ROWS_PW // C):
        pltpu.sync_copy(rows_v, acc_sh.at[pl.ds(base + i * C, C)])
    plsc.subcore_barrier()

    def blk(b, carry):
        pltpu.sync_copy(srcs_hbm.at[wid, pl.ds(b * IB, IB)], srcs_v)
        pltpu.sync_copy(dsts_hbm.at[wid, pl.ds(b * IB, IB)], dsts_v)

        def chunk(g, c1):
            pltpu.async_copy(haug_hbm.at[srcs_v.at[g]], rows_v, sem).wait()
            for q in range(C // 16):
                s16 = srcs_v[g, pl.ds(q * 16, 16)]
                d16 = dsts_v[g, pl.ds(q * 16, 16)]
                t = plsc.load_gather(asrc_v, [s16]) + plsc.load_gather(adst_v, [d16])
                t = jnp.maximum(t, 0.2 * t)
                wc_v[pl.ds(q * 16, 16)] = jnp.exp(t)

            def srow(j, c2):
                w = plsc.load_gather(wc_v, [jnp.full((16,), j, jnp.int32)])
                for k in range(HA // 16):
                    rows_v[j, pl.ds(k * 16, 16)] = rows_v[j, pl.ds(k * 16, 16)] * w
                return c2

            lax.fori_loop(0, C, srow, 0)
            pltpu.sync_copy(rows_v, acc_sh.at[dsts_v.at[g]], add=True)
            return c1

        lax.fori_loop(0, IB, chunk, 0)
        return carry

    lax.fori_loop(0, NCH // IB, blk, 0)
    plsc.subcore_barrier()
    pltpu.sync_copy(acc_sh.at[pl.ds(base, ROWS_PW)],
                    out_hbm.at[cid, pl.ds(base, ROWS_PW)])


def _sc_gat(h_aug, a_src, a_dst, srcs, dsts):
    mesh = plsc.VectorSubcoreMesh(core_axis_name="c", subcore_axis_name="s")
    return pl.kernel(
        _sc_gat_body,
        compiler_params=pltpu.CompilerParams(needs_layout_passes=False, use_tc_tiling_on_sc=False),
        out_type=jax.ShapeDtypeStruct((2, NP, HA), jnp.float32),
        mesh=mesh,
        scratch_types=[
            pltpu.VMEM((IB, C), jnp.int32),
            pltpu.VMEM((IB, C), jnp.int32),
            pltpu.VMEM((N,), jnp.float32),
            pltpu.VMEM((N,), jnp.float32),
            pltpu.VMEM((C, HA), jnp.float32),
            pltpu.VMEM((C,), jnp.float32),
            pltpu.VMEM_SHARED((NP, HA), jnp.float32),
            pltpu.SemaphoreType.DMA,
        ],
    )(h_aug, a_src, a_dst, srcs, dsts)


def _sc_dec_body(p_hbm, q_hbm, ridx_hbm, cidx_hbm, w2_hbm, out_hbm,
                 ridx_v, cidx_v, pbuf, qbuf, w2_v, tbuf, obuf, sem):
    cid = lax.axis_index("c")
    sid = lax.axis_index("s")
    wid = cid * 16 + sid

    pltpu.sync_copy(ridx_hbm.at[wid], ridx_v)
    pltpu.sync_copy(cidx_hbm.at[wid], cidx_v)
    pltpu.sync_copy(w2_hbm, w2_v)

    lane = jnp.arange(16, dtype=jnp.int32)

    def chunk(g, carry):
        cp = pltpu.async_copy(p_hbm.at[ridx_v.at[g]], pbuf, sem)
        cq = pltpu.async_copy(q_hbm.at[cidx_v.at[g]], qbuf, sem)
        cp.wait()
        cq.wait()

        def grp(j16, c2):
            for jj in range(16):
                acc = jnp.zeros((16,), jnp.float32)
                for k in range(H // 16):
                    pv = pbuf[j16 * 16 + jj, pl.ds(k * 16, 16)]
                    qv = qbuf[j16 * 16 + jj, pl.ds(k * 16, 16)]
                    acc = acc + jnp.maximum(pv + qv, 0.0) * w2_v[pl.ds(k * 16, 16)]
                tbuf[jj, :] = acc
            res = jnp.zeros((16,), jnp.float32)
            for k in range(16):
                res = res + plsc.load_gather(tbuf, [lane, jnp.full((16,), k, jnp.int32)])
            obuf[pl.ds(g * DC + j16 * 16, 16)] = res
            return c2

        lax.fori_loop(0, DC // 16, grp, 0)
        return carry

    lax.fori_loop(0, DNCH, chunk, 0)
    pltpu.sync_copy(obuf, out_hbm.at[pl.ds(wid * DPW, DPW)])


def _sc_dec(p, q, ridx, cidx, w2):
    mesh = plsc.VectorSubcoreMesh(core_axis_name="c", subcore_axis_name="s")
    return pl.kernel(
        _sc_dec_body,
        compiler_params=pltpu.CompilerParams(needs_layout_passes=False, use_tc_tiling_on_sc=False),
        out_type=jax.ShapeDtypeStruct((NW * DPW,), jnp.float32),
        mesh=mesh,
        scratch_types=[
            pltpu.VMEM((DNCH, DC), jnp.int32),
            pltpu.VMEM((DNCH, DC), jnp.int32),
            pltpu.VMEM((DC, H), jnp.float32),
            pltpu.VMEM((DC, H), jnp.float32),
            pltpu.VMEM((H,), jnp.float32),
            pltpu.VMEM((16, 16), jnp.float32),
            pltpu.VMEM((DPW,), jnp.float32),
            pltpu.SemaphoreType.DMA,
        ],
    )(p, q, ridx, cidx, w2)


def _aug_w(w_src):
    return jnp.concatenate([w_src, jnp.zeros((H, HA - H), jnp.float32)], axis=1)


_E_AUG = None


def kernel(x_sotu, x_taxon, params, node_id_sotu, node_id_taxon,
           edge_index_fwd, edge_index_rev, edge_label_index):
    p = params
    c1f, c1r, c2f, c2r = p["c1_fwd"], p["c1_rev"], p["c2_fwd"], p["c2_rev"]
    e_aug = jnp.zeros((1, HA), jnp.float32).at[0, H].set(1.0)

    sf = edge_index_fwd[0].astype(jnp.int32).reshape(NW, NCH, C)
    df = edge_index_fwd[1].astype(jnp.int32).reshape(NW, NCH, C)
    sr = edge_index_rev[0].astype(jnp.int32).reshape(NW, NCH, C)
    dr = edge_index_rev[1].astype(jnp.int32).reshape(NW, NCH, C)
    pad = NW * DPW - EL
    ridx = jnp.pad(edge_label_index[0].astype(jnp.int32), (0, pad)).reshape(NW, DNCH, DC)
    cidx = jnp.pad(edge_label_index[1].astype(jnp.int32), (0, pad)).reshape(NW, DNCH, DC)

    v_s1 = jnp.stack([c1f["W_src"] @ c1f["att_src"],
                      c1r["W_dst"] @ c1r["att_dst"]], axis=1)
    v_t1 = jnp.stack([c1f["W_dst"] @ c1f["att_dst"],
                      c1r["W_src"] @ c1r["att_src"]], axis=1)
    haug_s, a_s = _prep(x_sotu, p["sotu_emb"][node_id_sotu],
                        p["sotu_lin_W"], p["sotu_lin_b"].reshape(1, H),
                        _aug_w(c1f["W_src"]), e_aug, v_s1)
    haug_t, a_t = _prep(x_taxon, p["taxon_emb"][node_id_taxon],
                        p["taxon_lin_W"], p["taxon_lin_b"].reshape(1, H),
                        _aug_w(c1r["W_src"]), e_aug, v_t1)

    part_t = _sc_gat(haug_s, a_s[:, 0], a_t[:, 0], sf, df)
    part_s = _sc_gat(haug_t, a_t[:, 1], a_s[:, 1], sr, dr)

    v_s2 = jnp.stack([c2f["W_src"] @ c2f["att_src"],
                      c2r["W_dst"] @ c2r["att_dst"]], axis=1)
    v_t2 = jnp.stack([c2f["W_dst"] @ c2f["att_dst"],
                      c2r["W_src"] @ c2r["att_src"]], axis=1)
    haug2_f, a2_s = _mid(part_s, c1r["bias"].reshape(1, H),
                         _aug_w(c2f["W_src"]), e_aug, v_s2, relu=True)
    haug2_r, a2_t = _mid(part_t, c1f["bias"].reshape(1, H),
                         _aug_w(c2r["W_src"]), e_aug, v_t2, relu=True)

    part2_t = _sc_gat(haug2_f, a2_s[:, 0], a2_t[:, 0], sf, df)
    part2_s = _sc_gat(haug2_r, a2_t[:, 1], a2_s[:, 1], sr, dr)

    pw, _ = _mid(part2_s, c2r["bias"].reshape(1, H),
                 p["dec_W1"][:H], p["dec_b1"].reshape(1, H),
                 jnp.zeros((H, 2), jnp.float32), relu=False)
    qw, _ = _mid(part2_t, c2f["bias"].reshape(1, H),
                 p["dec_W1"][H:], jnp.zeros((1, H), jnp.float32),
                 jnp.zeros((H, 2), jnp.float32), relu=False)

    dec = _sc_dec(pw, qw, ridx, cidx, p["dec_W2"][:, 0])
    return dec[:EL] + p["dec_b2"][0]
